# baseline (speedup 1.0000x reference)
import functools
import math

import jax
import jax.numpy as jnp
from jax import lax
from jax.experimental import pallas as pl
from jax.experimental.pallas import tpu as pltpu
from jax.experimental.pallas import tpu_sc as plsc

VOCAB = 100000
D = 128
B = 4096
H = 20
HPAD = 24
NC, NS = 2, 16
NW = NC * NS
NSPLIT = 2
BSUB = B // NSPLIT
ROWS_W = BSUB // NW
RPC = 4
NCH = ROWS_W // RPC
GLEN = RPC * H
SLEN = RPC * HPAD
SCALE = float(math.sqrt(float(D)))

_mesh = plsc.VectorSubcoreMesh(core_axis_name="c", subcore_axis_name="s")


@functools.partial(
    pl.kernel,
    out_type=jax.ShapeDtypeStruct((BSUB, H, D), jnp.float32),
    mesh=_mesh,
    scratch_types=[
        pltpu.VMEM((ROWS_W * H,), jnp.int32),
        pltpu.VMEM((GLEN, D), jnp.float32),
        pltpu.VMEM((GLEN, D), jnp.float32),
        pltpu.VMEM((SLEN, D), jnp.float32),
        pltpu.VMEM((SLEN, D), jnp.float32),
        pltpu.SemaphoreType.DMA,
        pltpu.SemaphoreType.DMA,
        pltpu.SemaphoreType.DMA,
        pltpu.SemaphoreType.DMA,
    ],
    compiler_params=pltpu.CompilerParams(use_tc_tiling_on_sc=True),
)
def _embed_gather(idx_hbm, table_hbm, out_hbm, idx_v,
                  g_a, g_b, s_a, s_b, gsem_a, gsem_b, ssem_a, ssem_b):
    gbufs = (g_a, g_b)
    sbufs = (s_a, s_b)
    gsems = (gsem_a, gsem_b)
    ssems = (ssem_a, ssem_b)
    wid = lax.axis_index("s") * NC + lax.axis_index("c")
    b0 = wid * ROWS_W

    pltpu.sync_copy(idx_hbm.at[pl.ds(wid * ROWS_W * H, ROWS_W * H)], idx_v)

    pltpu.async_copy(table_hbm.at[idx_v.at[pl.ds(0, GLEN)]], gbufs[0], gsems[0])

    for j in range(NCH):
        p = j % 2
        gbuf, sbuf = gbufs[p], sbufs[p]
        pltpu.make_async_copy(
            table_hbm.at[idx_v.at[pl.ds(j * GLEN, GLEN)]], gbuf, gsems[p]
        ).wait()
        if j + 1 < NCH:
            pltpu.async_copy(
                table_hbm.at[idx_v.at[pl.ds((j + 1) * GLEN, GLEN)]],
                gbufs[1 - p], gsems[1 - p],
            )
        if j >= 2:
            for br in range(RPC):
                pltpu.make_async_copy(
                    sbuf.at[pl.ds(br * HPAD, H)],
                    out_hbm.at[b0 + (j - 2) * RPC + br],
                    ssems[p],
                ).wait()

        def scale_row(h, _, gbuf=gbuf, sbuf=sbuf):
            for br in range(RPC):
                for q in range(D // 16):
                    sbuf[br * HPAD + h, pl.ds(q * 16, 16)] = (
                        gbuf[br * H + h, pl.ds(q * 16, 16)] * SCALE)
            return 0

        lax.fori_loop(0, H, scale_row, 0)

        for br in range(RPC):
            pltpu.async_copy(
                sbuf.at[pl.ds(br * HPAD, H)],
                out_hbm.at[b0 + j * RPC + br],
                ssems[p],
            )

    for j in (NCH - 2, NCH - 1):
        p = j % 2
        for br in range(RPC):
            pltpu.make_async_copy(
                sbufs[p].at[pl.ds(br * HPAD, H)],
                out_hbm.at[b0 + j * RPC + br],
                ssems[p],
            ).wait()


def kernel(x, input_embedding_table):
    idx = x.astype(jnp.int32).reshape(NSPLIT, BSUB * H)
    parts = [_embed_gather(idx[i], input_embedding_table)
             for i in range(NSPLIT)]
    return jnp.concatenate(parts, axis=0)


# speedup vs baseline: 2.5211x; 2.5211x over previous
import functools
import math

import jax
import jax.numpy as jnp
from jax import lax
from jax.experimental import pallas as pl
from jax.experimental.pallas import tpu as pltpu
from jax.experimental.pallas import tpu_sc as plsc

VOCAB = 100000
D = 128
B = 4096
H = 20
NC, NS = 2, 16
NW = NC * NS
FLAT = B * H
PER_W = FLAT // NW
CHUNK = 128
NCH = PER_W // CHUNK
SCALE = float(math.sqrt(float(D)))

_mesh = plsc.VectorSubcoreMesh(core_axis_name="c", subcore_axis_name="s")


@functools.partial(
    pl.kernel,
    out_type=jax.ShapeDtypeStruct((FLAT, D), jnp.float32),
    mesh=_mesh,
    scratch_types=[
        pltpu.VMEM((PER_W,), jnp.int32),
        pltpu.VMEM((CHUNK, D), jnp.float32),
        pltpu.VMEM((CHUNK, D), jnp.float32),
        pltpu.SemaphoreType.DMA,
        pltpu.SemaphoreType.DMA,
    ],
    compiler_params=pltpu.CompilerParams(use_tc_tiling_on_sc=True),
)
def _embed_gather(idx_hbm, table_hbm, out_hbm, idx_v, buf_a, buf_b, sem_a, sem_b):
    bufs = (buf_a, buf_b)
    sems = (sem_a, sem_b)
    wid = lax.axis_index("s") * NC + lax.axis_index("c")
    base = wid * PER_W

    pltpu.sync_copy(idx_hbm.at[pl.ds(base, PER_W)], idx_v)

    pltpu.async_copy(table_hbm.at[idx_v.at[pl.ds(0, CHUNK)]], bufs[0], sems[0])

    for j in range(NCH):
        buf = bufs[j % 2]
        pltpu.make_async_copy(
            table_hbm.at[idx_v.at[pl.ds(j * CHUNK, CHUNK)]], buf, sems[j % 2]
        ).wait()
        if j + 1 < NCH:
            pltpu.async_copy(
                table_hbm.at[idx_v.at[pl.ds((j + 1) * CHUNK, CHUNK)]],
                bufs[(j + 1) % 2], sems[(j + 1) % 2],
            )

        def scale_row(r, _, buf=buf):
            for q in range(D // 16):
                buf[r, pl.ds(q * 16, 16)] = buf[r, pl.ds(q * 16, 16)] * SCALE
            return 0

        lax.fori_loop(0, CHUNK, scale_row, 0)
        pltpu.sync_copy(buf, out_hbm.at[pl.ds(base + j * CHUNK, CHUNK)])


def kernel(x, input_embedding_table):
    idx = x.astype(jnp.int32).T.reshape(FLAT)
    flat = _embed_gather(idx, input_embedding_table)
    return flat.reshape(H, B, D).transpose(1, 0, 2)


# speedup vs baseline: 2.8239x; 1.1201x over previous
import functools
import math

import jax
import jax.numpy as jnp
from jax import lax
from jax.experimental import pallas as pl
from jax.experimental.pallas import tpu as pltpu
from jax.experimental.pallas import tpu_sc as plsc

VOCAB = 100000
D = 128
B = 4096
H = 20
NC, NS = 2, 16
NW = NC * NS
FLAT = B * H
PER_W = FLAT // NW
CHUNK = 128
NCH = PER_W // CHUNK
SCALE = float(math.sqrt(float(D)))

_mesh = plsc.VectorSubcoreMesh(core_axis_name="c", subcore_axis_name="s")


@functools.partial(
    pl.kernel,
    out_type=jax.ShapeDtypeStruct((FLAT, D), jnp.float32),
    mesh=_mesh,
    scratch_types=[
        pltpu.VMEM((PER_W,), jnp.int32),
        *[pltpu.VMEM((CHUNK, D), jnp.float32) for _ in range(4)],
        *[pltpu.SemaphoreType.DMA for _ in range(8)],
    ],
    compiler_params=pltpu.CompilerParams(use_tc_tiling_on_sc=True),
)
def _embed_gather(idx_hbm, table_hbm, out_hbm, idx_v, *bufs_and_sems):
    bufs = bufs_and_sems[:4]
    gsems = bufs_and_sems[4:8]
    ssems = bufs_and_sems[8:]
    wid = lax.axis_index("s") * NC + lax.axis_index("c")
    base = wid * PER_W

    pltpu.sync_copy(idx_hbm.at[pl.ds(base, PER_W)], idx_v)

    pltpu.async_copy(table_hbm.at[idx_v.at[pl.ds(0, CHUNK)]], bufs[0], gsems[0])
    pltpu.async_copy(table_hbm.at[idx_v.at[pl.ds(CHUNK, CHUNK)]], bufs[1], gsems[1])

    for j in range(NCH):
        b = j % 4
        buf = bufs[b]
        pltpu.make_async_copy(
            table_hbm.at[idx_v.at[pl.ds(j * CHUNK, CHUNK)]], buf, gsems[b]
        ).wait()
        if j + 2 < NCH:
            nb = (j + 2) % 4
            if j >= 2:
                pltpu.make_async_copy(
                    bufs[nb], out_hbm.at[pl.ds(base + (j - 2) * CHUNK, CHUNK)],
                    ssems[nb],
                ).wait()
            pltpu.async_copy(
                table_hbm.at[idx_v.at[pl.ds((j + 2) * CHUNK, CHUNK)]],
                bufs[nb], gsems[nb],
            )

        def scale_row(r, _, buf=buf):
            for q in range(D // 16):
                buf[r, pl.ds(q * 16, 16)] = buf[r, pl.ds(q * 16, 16)] * SCALE
            return 0

        lax.fori_loop(0, CHUNK, scale_row, 0)
        pltpu.async_copy(buf, out_hbm.at[pl.ds(base + j * CHUNK, CHUNK)],
                         ssems[b])

    for j in range(NCH - 4, NCH):
        b = j % 4
        pltpu.make_async_copy(
            bufs[b], out_hbm.at[pl.ds(base + j * CHUNK, CHUNK)], ssems[b]
        ).wait()


def kernel(x, input_embedding_table):
    idx = x.astype(jnp.int32).T.reshape(FLAT)
    flat = _embed_gather(idx, input_embedding_table)
    return flat.reshape(H, B, D).transpose(1, 0, 2)


# speedup vs baseline: 2.9013x; 1.0274x over previous
import functools
import math

import jax
import jax.numpy as jnp
from jax import lax
from jax.experimental import pallas as pl
from jax.experimental.pallas import tpu as pltpu
from jax.experimental.pallas import tpu_sc as plsc

VOCAB = 100000
D = 128
B = 4096
H = 20
NC, NS = 2, 16
NW = NC * NS
FLAT = B * H
PER_W = FLAT // NW
CHUNK = 128
NCH = PER_W // CHUNK
SCALE = float(math.sqrt(float(D)))

_mesh = plsc.VectorSubcoreMesh(core_axis_name="c", subcore_axis_name="s")


@functools.partial(
    pl.kernel,
    out_type=jax.ShapeDtypeStruct((FLAT, D), jnp.float32),
    mesh=_mesh,
    scratch_types=[
        pltpu.VMEM((PER_W,), jnp.int32),
        *[pltpu.VMEM((CHUNK, D), jnp.float32) for _ in range(6)],
        *[pltpu.SemaphoreType.DMA for _ in range(12)],
    ],
    compiler_params=pltpu.CompilerParams(use_tc_tiling_on_sc=True),
)
def _embed_gather(idx_hbm, table_hbm, out_hbm, idx_v, *bufs_and_sems):
    bufs = bufs_and_sems[:6]
    gsems = bufs_and_sems[6:12]
    ssems = bufs_and_sems[12:]
    wid = lax.axis_index("s") * NC + lax.axis_index("c")
    base = wid * PER_W

    pltpu.sync_copy(idx_hbm.at[pl.ds(base, PER_W)], idx_v)

    for j in range(4):
        pltpu.async_copy(
            table_hbm.at[idx_v.at[pl.ds(j * CHUNK, CHUNK)]], bufs[j], gsems[j]
        )

    for j in range(NCH):
        b = j % 6
        buf = bufs[b]
        pltpu.make_async_copy(
            table_hbm.at[idx_v.at[pl.ds(j * CHUNK, CHUNK)]], buf, gsems[b]
        ).wait()
        if j + 4 < NCH:
            nb = (j + 4) % 6
            if j >= 2:
                pltpu.make_async_copy(
                    bufs[nb], out_hbm.at[pl.ds(base + (j - 2) * CHUNK, CHUNK)],
                    ssems[nb],
                ).wait()
            pltpu.async_copy(
                table_hbm.at[idx_v.at[pl.ds((j + 4) * CHUNK, CHUNK)]],
                bufs[nb], gsems[nb],
            )

        def scale_row(r, _, buf=buf):
            for q in range(D // 16):
                buf[r, pl.ds(q * 16, 16)] = buf[r, pl.ds(q * 16, 16)] * SCALE
            return 0

        lax.fori_loop(0, CHUNK, scale_row, 0)
        pltpu.async_copy(buf, out_hbm.at[pl.ds(base + j * CHUNK, CHUNK)],
                         ssems[b])

    for j in range(NCH - 6, NCH):
        b = j % 6
        pltpu.make_async_copy(
            bufs[b], out_hbm.at[pl.ds(base + j * CHUNK, CHUNK)], ssems[b]
        ).wait()


def kernel(x, input_embedding_table):
    idx = x.astype(jnp.int32).T.reshape(FLAT)
    flat = _embed_gather(idx, input_embedding_table)
    return flat.reshape(H, B, D).transpose(1, 0, 2)
